# baseline (speedup 1.0000x reference)
import jax
import jax.numpy as jnp
from jax import lax
from jax.experimental import pallas as pl
from jax.experimental.pallas import tpu as pltpu

_TOP_K = 2
_NEG_INF = float("-inf")


def _router_body(xc_ref, wr_ref, br_ref, wn_ref, bn_ref, noise_ref,
                 idx_ref, gate_ref):
    pooled = jnp.mean(xc_ref[...], axis=2)
    logits = jnp.dot(pooled, wr_ref[...],
                     preferred_element_type=jnp.float32) + br_ref[0]
    nlog = jnp.dot(pooled, wn_ref[...],
                   preferred_element_type=jnp.float32) + bn_ref[0]
    noisy = logits + noise_ref[...] * jax.nn.softplus(nlog)

    bs, ne = noisy.shape
    eids = lax.broadcasted_iota(jnp.int32, (bs, ne), 1)
    v0 = jnp.max(noisy, axis=1)
    i0 = jnp.min(jnp.where(noisy == v0[:, None], eids, ne), axis=1)
    masked = jnp.where(eids == i0[:, None], _NEG_INF, noisy)
    v1 = jnp.max(masked, axis=1)
    i1 = jnp.min(jnp.where(masked == v1[:, None], eids, ne), axis=1)
    t = jnp.exp(v1 - v0)
    g0 = 1.0 / (1.0 + t)
    g1 = t / (1.0 + t)
    idx_ref[...] = jnp.concatenate([i0[:, None], i1[:, None]], axis=1)
    gate_ref[...] = jnp.concatenate([g0[:, None], g1[:, None]], axis=1)


def _expert_body(idx_ref, gate_ref, xc_ref, w1a_ref, w1b_ref, b1a_ref,
                 b1b_ref, w2a_ref, w2b_ref, b2a_ref, b2b_ref, out_ref):
    b = pl.program_id(0)
    g0 = gate_ref[2 * b]
    g1 = gate_ref[2 * b + 1]
    xb = xc_ref[0]
    w1 = jnp.concatenate([w1a_ref[0], w1b_ref[0]], axis=1)
    b1 = jnp.concatenate([b1a_ref[0, 0], b1b_ref[0, 0]], axis=0)
    h1 = lax.dot_general(w1, xb, (((0,), (0,)), ((), ())),
                         preferred_element_type=jnp.float32)
    h1 = jnp.maximum(h1 + b1[:, None], 0.0)
    w2 = jnp.concatenate([g0 * w2a_ref[0], g1 * w2b_ref[0]], axis=0)
    h2 = lax.dot_general(w2, h1, (((0,), (0,)), ((), ())),
                         preferred_element_type=jnp.float32)
    out_ref[0] = h2 + (g0 * b2a_ref[0, 0] + g1 * b2b_ref[0, 0])[:, None]


def kernel(x, Wr, br, Wn, bn, W1, b1, W2, b2):
    bs, dim, h, w = x.shape
    hw = h * w
    ne = Wr.shape[1]
    hid = W1.shape[2]

    xc = x.reshape(bs, dim, hw)
    noise = jax.random.normal(jax.random.key(42), (bs, ne), dtype=jnp.float32)

    idx, gates = pl.pallas_call(
        _router_body,
        out_shape=(
            jax.ShapeDtypeStruct((bs, _TOP_K), jnp.int32),
            jax.ShapeDtypeStruct((bs, _TOP_K), jnp.float32),
        ),
    )(xc, Wr, br.reshape(1, ne), Wn, bn.reshape(1, ne), noise)

    idx_flat = idx.reshape(bs * _TOP_K)
    gates_flat = gates.reshape(bs * _TOP_K)

    def _e0(b, i_ref, g_ref):
        return (i_ref[2 * b], 0, 0)

    def _e1(b, i_ref, g_ref):
        return (i_ref[2 * b + 1], 0, 0)

    grid_spec = pltpu.PrefetchScalarGridSpec(
        num_scalar_prefetch=2,
        grid=(bs,),
        in_specs=[
            pl.BlockSpec((1, dim, hw), lambda b, i_ref, g_ref: (b, 0, 0)),
            pl.BlockSpec((1, dim, hid), _e0),
            pl.BlockSpec((1, dim, hid), _e1),
            pl.BlockSpec((1, 1, hid), _e0),
            pl.BlockSpec((1, 1, hid), _e1),
            pl.BlockSpec((1, hid, dim), _e0),
            pl.BlockSpec((1, hid, dim), _e1),
            pl.BlockSpec((1, 1, dim), _e0),
            pl.BlockSpec((1, 1, dim), _e1),
        ],
        out_specs=pl.BlockSpec((1, dim, hw), lambda b, i_ref, g_ref: (b, 0, 0)),
    )
    outp = pl.pallas_call(
        _expert_body,
        grid_spec=grid_spec,
        out_shape=jax.ShapeDtypeStruct((bs, dim, hw), jnp.float32),
    )(idx_flat, gates_flat, xc, W1, W1, b1.reshape(ne, 1, hid),
      b1.reshape(ne, 1, hid), W2, W2, b2.reshape(ne, 1, dim),
      b2.reshape(ne, 1, dim))

    return outp.reshape(bs, dim, h, w)
